# Initial kernel scaffold; baseline (speedup 1.0000x reference)
import jax
import jax.numpy as jnp
from jax.experimental import pallas as pl


def kernel(x, edge_index, W1, b1, W2, b2):
    raise NotImplementedError("write your pallas kernel here")


# speedup vs baseline: 8.2915x; 8.2915x over previous
import functools

import jax
import jax.numpy as jnp
from jax import lax
from jax.experimental import pallas as pl
from jax.experimental.pallas import tpu as pltpu
from jax.experimental.pallas import tpu_sc as plsc

N = 10000
NP = 10240
E = 320000
D = 128
HD = 64
K_PROP = 10
A = 0.1
B = 0.9

NS = 16
EP = E // NS
CH = 125
NCHUNK = EP // CH
RP = NP // NS
RCH = 128
NRCH = RP // RCH


def _mlp_split(x, W1, b1, W2, b2):
    RB = 400
    grid = (N // RB,)

    def body(x_ref, w1_ref, b1_ref, w2_ref, b2_ref, h_ref):
        h1 = jnp.maximum(
            jnp.dot(x_ref[...], w1_ref[...], preferred_element_type=jnp.float32)
            + b1_ref[...], 0.0)
        h2 = (jnp.dot(h1, w2_ref[...], preferred_element_type=jnp.float32)
              + b2_ref[...])
        h_ref[0] = h2[:, :HD]
        h_ref[1] = h2[:, HD:]

    return pl.pallas_call(
        body,
        grid=grid,
        in_specs=[
            pl.BlockSpec((RB, D), lambda i: (i, 0)),
            pl.BlockSpec((D, D), lambda i: (0, 0)),
            pl.BlockSpec((1, D), lambda i: (0, 0)),
            pl.BlockSpec((D, D), lambda i: (0, 0)),
            pl.BlockSpec((1, D), lambda i: (0, 0)),
        ],
        out_specs=pl.BlockSpec((2, RB, HD), lambda i: (0, i, 0)),
        out_shape=jax.ShapeDtypeStruct((2, NP, HD), jnp.float32),
    )(x, W1, b1.reshape(1, D), W2, b2.reshape(1, D))


def _rsqrt16(x):
    i = plsc.bitcast(x, jnp.int32)
    i = 0x5F3759DF - lax.shift_right_logical(i, 1)
    y = plsc.bitcast(i, jnp.float32)
    for _ in range(3):
        y = y * (1.5 - 0.5 * x * y * y)
    return y


def _make_prop():
    mesh = plsc.VectorSubcoreMesh(core_axis_name="c", subcore_axis_name="s")

    @functools.partial(
        pl.kernel,
        mesh=mesh,
        compiler_params=pltpu.CompilerParams(
            needs_layout_passes=False, use_tc_tiling_on_sc=False),
        out_type=jax.ShapeDtypeStruct((2 * NP, HD), jnp.float32),
        scratch_types=[
            pltpu.VMEM_SHARED((NP, HD), jnp.float32),
            pltpu.VMEM_SHARED((NP, HD), jnp.float32),
            pltpu.VMEM((2, CH), jnp.int32),
            pltpu.VMEM((2, CH), jnp.int32),
            pltpu.VMEM((CH, HD), jnp.float32),
            pltpu.VMEM((RCH, HD), jnp.float32),
            pltpu.VMEM((RCH, HD), jnp.float32),
            pltpu.VMEM((RCH, HD), jnp.float32),
            pltpu.VMEM((RP, 16), jnp.float32),
        ],
    )
    def prop(h_hbm, src_hbm, dst_hbm, out_hbm,
             y_sp, s_sp, src_v, dst_v, rows_v, s_c, y_c, h_c, dis_v):
        cid = lax.axis_index("c")
        sid = lax.axis_index("s")
        row0 = sid * RP
        hbase = cid * NP

        zero16 = jnp.zeros((16,), jnp.float32)

        def fill_rows(ref, nrows, val):
            def row(r, _):
                for u in range(HD // 16):
                    ref[r, pl.ds(16 * u, 16)] = jnp.full((16,), val, jnp.float32)
                return 0
            lax.fori_loop(0, nrows, row, 0)

        fill_rows(h_c, RCH, 0.0)
        for q in range(NRCH):
            pltpu.sync_copy(h_c, s_sp.at[pl.ds(row0 + RCH * q, RCH)])
        plsc.subcore_barrier()

        fill_rows(rows_v, CH, 1.0)

        def deg_chunk(j, _):
            pltpu.sync_copy(dst_hbm.at[sid, j], dst_v.at[0])
            pltpu.sync_copy(rows_v, s_sp.at[dst_v.at[0]], add=True)
            return 0
        lax.fori_loop(0, NCHUNK, deg_chunk, 0)
        plsc.subcore_barrier()

        for q in range(NRCH):
            rbase = row0 + RCH * q
            pltpu.sync_copy(s_sp.at[pl.ds(rbase, RCH)], s_c)
            pltpu.sync_copy(h_hbm.at[pl.ds(hbase + rbase, RCH)], h_c)

            def initrow(r, _, q=q):
                deg = s_c[r, pl.ds(0, 16)] + 1.0
                dis = _rsqrt16(deg)
                dis_v[RCH * q + r, :] = dis
                for u in range(HD // 16):
                    sl = pl.ds(16 * u, 16)
                    y_c[r, sl] = dis * h_c[r, sl]
                    s_c[r, sl] = zero16
                return 0
            lax.fori_loop(0, RCH, initrow, 0)
            pltpu.sync_copy(y_c, y_sp.at[pl.ds(rbase, RCH)])
            pltpu.sync_copy(s_c, s_sp.at[pl.ds(rbase, RCH)])
        plsc.subcore_barrier()

        def edge_phase():
            def echunk(j, _):
                pltpu.sync_copy(src_hbm.at[sid, j], src_v.at[0])
                pltpu.sync_copy(dst_hbm.at[sid, j], dst_v.at[0])
                pltpu.sync_copy(y_sp.at[src_v.at[0]], rows_v)
                pltpu.sync_copy(rows_v, s_sp.at[dst_v.at[0]], add=True)
                return 0
            lax.fori_loop(0, NCHUNK, echunk, 0)
            plsc.subcore_barrier()

        def iter_body(k, _):
            edge_phase()
            for q in range(NRCH):
                rbase = row0 + RCH * q
                pltpu.sync_copy(s_sp.at[pl.ds(rbase, RCH)], s_c)
                pltpu.sync_copy(y_sp.at[pl.ds(rbase, RCH)], y_c)
                pltpu.sync_copy(h_hbm.at[pl.ds(hbase + rbase, RCH)], h_c)

                def rowfn(r, _2, q=q):
                    dis = dis_v[RCH * q + r, :]
                    bd2 = B * dis * dis
                    ad = A * dis
                    for u in range(HD // 16):
                        sl = pl.ds(16 * u, 16)
                        y_c[r, sl] = (bd2 * (s_c[r, sl] + y_c[r, sl])
                                      + ad * h_c[r, sl])
                        s_c[r, sl] = zero16
                    return 0
                lax.fori_loop(0, RCH, rowfn, 0)
                pltpu.sync_copy(y_c, y_sp.at[pl.ds(rbase, RCH)])
                pltpu.sync_copy(s_c, s_sp.at[pl.ds(rbase, RCH)])
            plsc.subcore_barrier()
            return 0
        lax.fori_loop(0, K_PROP - 1, iter_body, 0)

        edge_phase()
        for q in range(NRCH):
            rbase = row0 + RCH * q
            pltpu.sync_copy(s_sp.at[pl.ds(rbase, RCH)], s_c)
            pltpu.sync_copy(y_sp.at[pl.ds(rbase, RCH)], y_c)
            pltpu.sync_copy(h_hbm.at[pl.ds(hbase + rbase, RCH)], h_c)

            def finrow(r, _, q=q):
                dis = dis_v[RCH * q + r, :]
                bd = B * dis
                for u in range(HD // 16):
                    sl = pl.ds(16 * u, 16)
                    y_c[r, sl] = (bd * (s_c[r, sl] + y_c[r, sl])
                                  + A * h_c[r, sl])
                return 0
            lax.fori_loop(0, RCH, finrow, 0)
            pltpu.sync_copy(y_c, out_hbm.at[pl.ds(hbase + rbase, RCH)])

    return prop


_prop = _make_prop()


def kernel(x, edge_index, W1, b1, W2, b2):
    h_split = _mlp_split(x, W1, b1, W2, b2)
    h2d = h_split.reshape(2 * NP, HD)
    src = edge_index[0].reshape(NS, NCHUNK, CH)
    dst = edge_index[1].reshape(NS, NCHUNK, CH)
    o = _prop(h2d, src, dst)
    return jnp.concatenate([o[:N], o[NP:NP + N]], axis=1)


# speedup vs baseline: 17.4042x; 2.0990x over previous
import functools

import jax
import jax.numpy as jnp
from jax import lax
from jax.experimental import pallas as pl
from jax.experimental.pallas import tpu as pltpu
from jax.experimental.pallas import tpu_sc as plsc

N = 10000
NP = 10240
E = 320000
D = 128
HD = 64
K_PROP = 10
A = 0.1
B = 0.9

NS = 16
EP = E // NS
CH = 125
NCHUNK = EP // CH
RP = NP // NS
RCH = 64
NRCH = RP // RCH


def _mlp_split(x, W1, b1, W2, b2):
    RB = 400
    grid = (N // RB,)

    def body(x_ref, w1_ref, b1_ref, w2_ref, b2_ref, h_ref):
        h1 = jnp.maximum(
            jnp.dot(x_ref[...], w1_ref[...], preferred_element_type=jnp.float32)
            + b1_ref[...], 0.0)
        h2 = (jnp.dot(h1, w2_ref[...], preferred_element_type=jnp.float32)
              + b2_ref[...])
        h_ref[0] = h2[:, :HD]
        h_ref[1] = h2[:, HD:]

    return pl.pallas_call(
        body,
        grid=grid,
        in_specs=[
            pl.BlockSpec((RB, D), lambda i: (i, 0)),
            pl.BlockSpec((D, D), lambda i: (0, 0)),
            pl.BlockSpec((1, D), lambda i: (0, 0)),
            pl.BlockSpec((D, D), lambda i: (0, 0)),
            pl.BlockSpec((1, D), lambda i: (0, 0)),
        ],
        out_specs=pl.BlockSpec((2, RB, HD), lambda i: (0, i, 0)),
        out_shape=jax.ShapeDtypeStruct((2, NP, HD), jnp.float32),
    )(x, W1, b1.reshape(1, D), W2, b2.reshape(1, D))


def _rsqrt16(x):
    i = plsc.bitcast(x, jnp.int32)
    i = 0x5F3759DF - lax.shift_right_logical(i, 1)
    y = plsc.bitcast(i, jnp.float32)
    for _ in range(3):
        y = y * (1.5 - 0.5 * x * y * y)
    return y


def _make_prop():
    mesh = plsc.VectorSubcoreMesh(core_axis_name="c", subcore_axis_name="s")

    @functools.partial(
        pl.kernel,
        mesh=mesh,
        compiler_params=pltpu.CompilerParams(
            needs_layout_passes=False, use_tc_tiling_on_sc=False),
        out_type=jax.ShapeDtypeStruct((2 * NP, HD), jnp.float32),
        scratch_types=[
            pltpu.VMEM_SHARED((NP, HD), jnp.float32),
            pltpu.VMEM_SHARED((NP, HD), jnp.float32),
            pltpu.VMEM((4, CH), jnp.int32),
            pltpu.VMEM((4, CH), jnp.int32),
            pltpu.VMEM((2, CH, HD), jnp.float32),
            pltpu.VMEM((RCH, HD), jnp.float32),
            pltpu.VMEM((RCH, HD), jnp.float32),
            pltpu.VMEM((RCH, HD), jnp.float32),
            pltpu.VMEM((RP, 16), jnp.float32),
            pltpu.SemaphoreType.DMA,
            pltpu.SemaphoreType.DMA,
            pltpu.SemaphoreType.DMA,
        ],
    )
    def prop(h_hbm, src_hbm, dst_hbm, out_hbm,
             y_sp, s_sp, src_v, dst_v, rows_v, s_c, y_c, h_c, dis_v,
             sem_i, sem_g, sem_s):
        cid = lax.axis_index("c")
        sid = lax.axis_index("s")
        row0 = sid * RP
        hbase = cid * NP

        zero16 = jnp.zeros((16,), jnp.float32)

        def fill_rows(ref, nrows, val):
            def row(r, _):
                for u in range(HD // 16):
                    ref[r, pl.ds(16 * u, 16)] = jnp.full((16,), val, jnp.float32)
                return 0
            lax.fori_loop(0, nrows, row, 0)

        fill_rows(h_c, RCH, 0.0)
        for q in range(NRCH):
            pltpu.sync_copy(h_c, s_sp.at[pl.ds(row0 + RCH * q, RCH)])
        plsc.subcore_barrier()

        def fill_ones(r, _):
            for u in range(HD // 16):
                rows_v[0, r, pl.ds(16 * u, 16)] = jnp.full((16,), 1.0,
                                                           jnp.float32)
            return 0
        lax.fori_loop(0, CH, fill_ones, 0)

        for cc in range(2):
            pltpu.async_copy(dst_hbm.at[sid, cc], dst_v.at[cc], sem_i)

        @pl.loop(0, NCHUNK // 4)
        def _deg(jj):
            for b in range(4):
                c = 4 * jj + b
                pltpu.make_async_copy(dst_hbm.at[sid, c], dst_v.at[b],
                                      sem_i).wait()

                @pl.when(c >= 2)
                def _():
                    pltpu.make_async_copy(rows_v.at[0],
                                          s_sp.at[dst_v.at[b]], sem_s).wait()
                pltpu.async_copy(rows_v.at[0], s_sp.at[dst_v.at[b]], sem_s,
                                 add=True)
                cn = jnp.minimum(c + 2, NCHUNK - 1)
                pltpu.async_copy(dst_hbm.at[sid, cn],
                                 dst_v.at[(b + 2) % 4], sem_i)
        for b in range(2):
            pltpu.make_async_copy(dst_hbm.at[sid, 0], dst_v.at[b],
                                  sem_i).wait()
            pltpu.make_async_copy(rows_v.at[0], s_sp.at[dst_v.at[b]],
                                  sem_s).wait()
        plsc.subcore_barrier()

        for q in range(NRCH):
            rbase = row0 + RCH * q
            pltpu.sync_copy(s_sp.at[pl.ds(rbase, RCH)], s_c)
            pltpu.sync_copy(h_hbm.at[pl.ds(hbase + rbase, RCH)], h_c)

            def initrow(r, _, q=q):
                deg = s_c[r, pl.ds(0, 16)] + 1.0
                dis = _rsqrt16(deg)
                dis_v[RCH * q + r, :] = dis
                for u in range(HD // 16):
                    sl = pl.ds(16 * u, 16)
                    y_c[r, sl] = dis * h_c[r, sl]
                    s_c[r, sl] = zero16
                return 0
            lax.fori_loop(0, RCH, initrow, 0)
            pltpu.sync_copy(y_c, y_sp.at[pl.ds(rbase, RCH)])
            pltpu.sync_copy(s_c, s_sp.at[pl.ds(rbase, RCH)])
        plsc.subcore_barrier()

        def edge_phase():
            for cc in range(2):
                pltpu.async_copy(src_hbm.at[sid, cc], src_v.at[cc], sem_i)
                pltpu.async_copy(dst_hbm.at[sid, cc], dst_v.at[cc], sem_i)

            @pl.loop(0, NCHUNK // 4)
            def _edges(jj):
                for b in range(4):
                    c = 4 * jj + b
                    s2 = b % 2
                    pltpu.make_async_copy(src_hbm.at[sid, c], src_v.at[b],
                                          sem_i).wait()
                    pltpu.make_async_copy(dst_hbm.at[sid, c], dst_v.at[b],
                                          sem_i).wait()

                    @pl.when(c >= 2)
                    def _():
                        pltpu.make_async_copy(rows_v.at[s2],
                                              s_sp.at[dst_v.at[b]],
                                              sem_s).wait()
                    pltpu.async_copy(y_sp.at[src_v.at[b]], rows_v.at[s2],
                                     sem_g).wait()
                    pltpu.async_copy(rows_v.at[s2], s_sp.at[dst_v.at[b]],
                                     sem_s, add=True)
                    cn = jnp.minimum(c + 2, NCHUNK - 1)
                    pltpu.async_copy(src_hbm.at[sid, cn],
                                     src_v.at[(b + 2) % 4], sem_i)
                    pltpu.async_copy(dst_hbm.at[sid, cn],
                                     dst_v.at[(b + 2) % 4], sem_i)
            for b in range(2):
                pltpu.make_async_copy(src_hbm.at[sid, 0], src_v.at[b],
                                      sem_i).wait()
                pltpu.make_async_copy(dst_hbm.at[sid, 0], dst_v.at[b],
                                      sem_i).wait()
                pltpu.make_async_copy(rows_v.at[b], s_sp.at[dst_v.at[b]],
                                      sem_s).wait()
            plsc.subcore_barrier()

        def iter_body(k, _):
            edge_phase()
            for q in range(NRCH):
                rbase = row0 + RCH * q
                pltpu.sync_copy(s_sp.at[pl.ds(rbase, RCH)], s_c)
                pltpu.sync_copy(y_sp.at[pl.ds(rbase, RCH)], y_c)
                pltpu.sync_copy(h_hbm.at[pl.ds(hbase + rbase, RCH)], h_c)

                def rowfn(r, _2, q=q):
                    dis = dis_v[RCH * q + r, :]
                    bd2 = B * dis * dis
                    ad = A * dis
                    for u in range(HD // 16):
                        sl = pl.ds(16 * u, 16)
                        y_c[r, sl] = (bd2 * (s_c[r, sl] + y_c[r, sl])
                                      + ad * h_c[r, sl])
                        s_c[r, sl] = zero16
                    return 0
                lax.fori_loop(0, RCH, rowfn, 0)
                pltpu.sync_copy(y_c, y_sp.at[pl.ds(rbase, RCH)])
                pltpu.sync_copy(s_c, s_sp.at[pl.ds(rbase, RCH)])
            plsc.subcore_barrier()
            return 0
        lax.fori_loop(0, K_PROP - 1, iter_body, 0)

        edge_phase()
        for q in range(NRCH):
            rbase = row0 + RCH * q
            pltpu.sync_copy(s_sp.at[pl.ds(rbase, RCH)], s_c)
            pltpu.sync_copy(y_sp.at[pl.ds(rbase, RCH)], y_c)
            pltpu.sync_copy(h_hbm.at[pl.ds(hbase + rbase, RCH)], h_c)

            def finrow(r, _, q=q):
                dis = dis_v[RCH * q + r, :]
                bd = B * dis
                for u in range(HD // 16):
                    sl = pl.ds(16 * u, 16)
                    y_c[r, sl] = (bd * (s_c[r, sl] + y_c[r, sl])
                                  + A * h_c[r, sl])
                return 0
            lax.fori_loop(0, RCH, finrow, 0)
            pltpu.sync_copy(y_c, out_hbm.at[pl.ds(hbase + rbase, RCH)])

    return prop


_prop = _make_prop()


def kernel(x, edge_index, W1, b1, W2, b2):
    h_split = _mlp_split(x, W1, b1, W2, b2)
    h2d = h_split.reshape(2 * NP, HD)
    src = edge_index[0].reshape(NS, NCHUNK, CH)
    dst = edge_index[1].reshape(NS, NCHUNK, CH)
    o = _prop(h2d, src, dst)
    return jnp.concatenate([o[:N], o[NP:NP + N]], axis=1)
